# baseline (speedup 1.0000x reference)
import jax
import jax.numpy as jnp
from jax import lax
from jax.experimental import pallas as pl
from jax.experimental.pallas import tpu as pltpu
from jax.experimental.pallas import tpu_sc as plsc

RING = 4
LOOKAHEAD = 3


def _make_kernel(B, V, E, S):
    info = plsc.get_sparse_core_info()
    NC, NS, L = info.num_cores, info.num_subcores, info.num_lanes
    NW = NC * NS
    CH = S
    assert B % (NW * CH) == 0 and E % L == 0
    BPW = B // NW
    NCH = BPW // CH

    mesh = plsc.VectorSubcoreMesh(core_axis_name="c", subcore_axis_name="s")

    def body(x_hbm, wt_hbm, pt_hbm, out_hbm, idx_all, pos_v, rows, gsems, wsems):
        wid = lax.axis_index("s") * NC + lax.axis_index("c")
        base = wid * BPW

        pltpu.sync_copy(x_hbm.at[pl.ds(wid * NCH, NCH), :], idx_all)
        pltpu.sync_copy(pt_hbm, pos_v)

        def gather(chunk, b):
            return pltpu.make_async_copy(
                wt_hbm.at[idx_all.at[chunk]], rows[b], gsems[b])

        def write(chunk, b):
            off = pl.multiple_of(base + chunk * CH, CH)
            return pltpu.make_async_copy(
                rows[b], out_hbm.at[pl.ds(off, CH)], wsems[b])

        for b in range(LOOKAHEAD):
            gather(b, b).start()

        @pl.loop(0, NCH, step=RING)
        def _(g):
            for b in range(RING):
                c = g + b
                gather(c, b).wait()
                p = c + LOOKAHEAD

                @pl.when(p < NCH)
                def _():
                    pb = (b + LOOKAHEAD) % RING

                    @pl.when(p >= RING)
                    def _():
                        write(p - RING, pb).wait()

                    gather(p, pb).start()

                @pl.loop(0, CH, unroll=8)
                def _(r):
                    for col in range(E // L):
                        sl = pl.ds(col * L, L)
                        plsc.addupdate(rows[b].at[r, sl], pos_v[r, sl])

                write(c, b).start()

        for b in range(RING):
            write(NCH - RING + b, b).wait()

    return pl.kernel(
        body,
        out_type=jax.ShapeDtypeStruct((B, E), jnp.float32),
        mesh=mesh,
        scratch_types=[
            pltpu.VMEM((NCH, CH), jnp.int32),
            pltpu.VMEM((S, E), jnp.float32),
            [pltpu.VMEM((CH, E), jnp.float32)] * RING,
            [pltpu.SemaphoreType.DMA] * RING,
            [pltpu.SemaphoreType.DMA] * RING,
        ],
    )


@jax.jit
def kernel(x, word_table, pos_table):
    N, S = x.shape
    V, E = word_table.shape
    flat = _make_kernel(N * S, V, E, S)(
        x.astype(jnp.int32), word_table, pos_table
    )
    return flat.reshape(N, S, E)


# speedup vs baseline: 1.1762x; 1.1762x over previous
import jax
import jax.numpy as jnp
from jax import lax
from jax.experimental import pallas as pl
from jax.experimental.pallas import tpu as pltpu
from jax.experimental.pallas import tpu_sc as plsc

CH = 64
RING = 8
LOOKAHEAD = 4


def _make_kernel(B, V, E, S):
    info = plsc.get_sparse_core_info()
    NC, NS, L = info.num_cores, info.num_subcores, info.num_lanes
    NW = NC * NS
    assert B % (NW * CH) == 0 and E % L == 0 and S % CH == 0
    assert (RING * CH) % S == 0
    POS_PER = S // CH
    BPW = B // NW
    NCH = BPW // CH
    assert NCH % RING == 0

    mesh = plsc.VectorSubcoreMesh(core_axis_name="c", subcore_axis_name="s")

    def body(x_hbm, wt_hbm, pt_hbm, out_hbm, idx_all, pos_v, rows, gsems, wsems):
        wid = lax.axis_index("s") * NC + lax.axis_index("c")
        base = wid * BPW

        pltpu.sync_copy(x_hbm.at[pl.ds(wid * NCH, NCH), :], idx_all)
        pltpu.sync_copy(pt_hbm, pos_v)

        def gather(chunk, b):
            return pltpu.make_async_copy(
                wt_hbm.at[idx_all.at[chunk]], rows[b], gsems[b])

        def write(chunk, b):
            off = pl.multiple_of(base + chunk * CH, CH)
            return pltpu.make_async_copy(
                rows[b], out_hbm.at[pl.ds(off, CH)], wsems[b])

        for b in range(LOOKAHEAD):
            gather(b, b).start()

        @pl.loop(0, NCH, step=RING)
        def _(g):
            for b in range(RING):
                c = g + b
                gather(c, b).wait()
                p = c + LOOKAHEAD

                @pl.when(p < NCH)
                def _():
                    pb = (b + LOOKAHEAD) % RING

                    @pl.when(p >= RING)
                    def _():
                        write(p - RING, pb).wait()

                    gather(p, pb).start()

                pbase = (b % POS_PER) * CH

                @pl.loop(0, CH, unroll=8)
                def _(r):
                    for col in range(E // L):
                        sl = pl.ds(col * L, L)
                        plsc.addupdate(rows[b].at[r, sl], pos_v[pbase + r, sl])

                write(c, b).start()

        for b in range(RING):
            write(NCH - RING + b, b).wait()

    return pl.kernel(
        body,
        out_type=jax.ShapeDtypeStruct((B, E), jnp.float32),
        mesh=mesh,
        scratch_types=[
            pltpu.VMEM((B // NW // CH, CH), jnp.int32),
            pltpu.VMEM((S, E), jnp.float32),
            [pltpu.VMEM((CH, E), jnp.float32)] * RING,
            [pltpu.SemaphoreType.DMA] * RING,
            [pltpu.SemaphoreType.DMA] * RING,
        ],
    )


@jax.jit
def kernel(x, word_table, pos_table):
    N, S = x.shape
    V, E = word_table.shape
    B = N * S
    flat = _make_kernel(B, V, E, S)(
        x.reshape(B // CH, CH).astype(jnp.int32), word_table, pos_table
    )
    return flat.reshape(N, S, E)


# speedup vs baseline: 1.1824x; 1.0053x over previous
import jax
import jax.numpy as jnp
from jax import lax
from jax.experimental import pallas as pl
from jax.experimental.pallas import tpu as pltpu
from jax.experimental.pallas import tpu_sc as plsc

CH = 64
RING = 8
LOOKAHEAD = 6


def _make_kernel(B, V, E, S):
    info = plsc.get_sparse_core_info()
    NC, NS, L = info.num_cores, info.num_subcores, info.num_lanes
    NW = NC * NS
    assert B % (NW * CH) == 0 and E % L == 0 and S % CH == 0
    assert (RING * CH) % S == 0
    POS_PER = S // CH
    BPW = B // NW
    NCH = BPW // CH
    assert NCH % RING == 0

    mesh = plsc.VectorSubcoreMesh(core_axis_name="c", subcore_axis_name="s")

    def body(x_hbm, wt_hbm, pt_hbm, out_hbm, idx_all, pos_v, rows, gsems, wsems):
        wid = lax.axis_index("s") * NC + lax.axis_index("c")
        base = wid * BPW

        pltpu.sync_copy(x_hbm.at[pl.ds(wid * NCH, NCH), :], idx_all)
        pltpu.sync_copy(pt_hbm, pos_v)

        def gather(chunk, b):
            return pltpu.make_async_copy(
                wt_hbm.at[idx_all.at[chunk]], rows[b], gsems[b])

        def write(chunk, b):
            off = pl.multiple_of(base + chunk * CH, CH)
            return pltpu.make_async_copy(
                rows[b], out_hbm.at[pl.ds(off, CH)], wsems[b])

        for b in range(LOOKAHEAD):
            gather(b, b).start()

        @pl.loop(0, NCH, step=RING)
        def _(g):
            for b in range(RING):
                c = g + b
                gather(c, b).wait()
                p = c + LOOKAHEAD

                @pl.when(p < NCH)
                def _():
                    pb = (b + LOOKAHEAD) % RING

                    @pl.when(p >= RING)
                    def _():
                        write(p - RING, pb).wait()

                    gather(p, pb).start()

                pbase = (b % POS_PER) * CH

                @pl.loop(0, CH, unroll=8)
                def _(r):
                    for col in range(E // L):
                        sl = pl.ds(col * L, L)
                        plsc.addupdate(rows[b].at[r, sl], pos_v[pbase + r, sl])

                write(c, b).start()

        for b in range(RING):
            write(NCH - RING + b, b).wait()

    return pl.kernel(
        body,
        out_type=jax.ShapeDtypeStruct((B, E), jnp.float32),
        mesh=mesh,
        scratch_types=[
            pltpu.VMEM((B // NW // CH, CH), jnp.int32),
            pltpu.VMEM((S, E), jnp.float32),
            [pltpu.VMEM((CH, E), jnp.float32)] * RING,
            [pltpu.SemaphoreType.DMA] * RING,
            [pltpu.SemaphoreType.DMA] * RING,
        ],
    )


@jax.jit
def kernel(x, word_table, pos_table):
    N, S = x.shape
    V, E = word_table.shape
    B = N * S
    flat = _make_kernel(B, V, E, S)(
        x.reshape(B // CH, CH).astype(jnp.int32), word_table, pos_table
    )
    return flat.reshape(N, S, E)


# speedup vs baseline: 1.2357x; 1.0451x over previous
import jax
import jax.numpy as jnp
from jax import lax
from jax.experimental import pallas as pl
from jax.experimental.pallas import tpu as pltpu
from jax.experimental.pallas import tpu_sc as plsc

RING = 4
LOOKAHEAD = 2


def _make_kernel(B, V, E, S):
    info = plsc.get_sparse_core_info()
    NC, NS, L = info.num_cores, info.num_subcores, info.num_lanes
    NW = NC * NS
    CH = S
    assert B % (NW * CH) == 0 and E % L == 0
    BPW = B // NW
    NCH = BPW // CH

    mesh = plsc.VectorSubcoreMesh(core_axis_name="c", subcore_axis_name="s")

    def body(x_hbm, wt_hbm, pt_hbm, out_hbm, idx_all, pos_v, rows, gsems, wsems):
        wid = lax.axis_index("s") * NC + lax.axis_index("c")
        base = wid * BPW

        idx_cp = pltpu.make_async_copy(
            x_hbm.at[pl.ds(wid * NCH, NCH), :], idx_all, wsems[0])
        pos_cp = pltpu.make_async_copy(pt_hbm, pos_v, wsems[1])
        idx_cp.start()
        pos_cp.start()

        def gather(chunk, b):
            return pltpu.make_async_copy(
                wt_hbm.at[idx_all.at[chunk]], rows[b], gsems[b])

        def write(chunk, b):
            off = pl.multiple_of(base + chunk * CH, CH)
            return pltpu.make_async_copy(
                rows[b], out_hbm.at[pl.ds(off, CH)], wsems[b])

        idx_cp.wait()
        for b in range(LOOKAHEAD):
            gather(b, b).start()
        pos_cp.wait()

        @pl.loop(0, NCH, step=RING)
        def _(g):
            for b in range(RING):
                c = g + b
                gather(c, b).wait()
                p = c + LOOKAHEAD

                @pl.when(p < NCH)
                def _():
                    pb = (b + LOOKAHEAD) % RING

                    @pl.when(p >= RING)
                    def _():
                        write(p - RING, pb).wait()

                    gather(p, pb).start()

                @pl.loop(0, CH, unroll=8)
                def _(r):
                    for col in range(E // L):
                        sl = pl.ds(col * L, L)
                        plsc.addupdate(rows[b].at[r, sl], pos_v[r, sl])

                write(c, b).start()

        for b in range(RING):
            write(NCH - RING + b, b).wait()

    return pl.kernel(
        body,
        out_type=jax.ShapeDtypeStruct((B, E), jnp.float32),
        mesh=mesh,
        scratch_types=[
            pltpu.VMEM((NCH, CH), jnp.int32),
            pltpu.VMEM((S, E), jnp.float32),
            [pltpu.VMEM((CH, E), jnp.float32)] * RING,
            [pltpu.SemaphoreType.DMA] * RING,
            [pltpu.SemaphoreType.DMA] * RING,
        ],
    )


@jax.jit
def kernel(x, word_table, pos_table):
    N, S = x.shape
    V, E = word_table.shape
    flat = _make_kernel(N * S, V, E, S)(
        x.astype(jnp.int32), word_table, pos_table
    )
    return flat.reshape(N, S, E)
